# Initial kernel scaffold; baseline (speedup 1.0000x reference)
import jax
import jax.numpy as jnp
from jax.experimental import pallas as pl


def kernel(nodes_u, nodes_i, is_user, friends_table, ufeat, ifeat, u_W, u_b, i_W, i_b, uf_W, uf_b, if_W, if_b, l2_W, l2_b, K, a1_W, a1_b, a2_W, a2_b, a3_W, a3_b):
    raise NotImplementedError("write your pallas kernel here")


# speedup vs baseline: 4.6759x; 4.6759x over previous
import functools

import jax
import jax.numpy as jnp
from jax import lax
from jax.experimental import pallas as pl
from jax.experimental.pallas import tpu as pltpu
from jax.experimental.pallas import tpu_sc as plsc

B = 4096
F = 32
FEAT = 256
D = 128
BN = 128
G = B + B * F
NW = 32
PER_W = G // NW
CHUNK = 128
NCH = PER_W // CHUNK


def _sc_gather(all_idx, feat):
    mesh = plsc.VectorSubcoreMesh(core_axis_name="c", subcore_axis_name="s")

    @functools.partial(
        pl.kernel,
        mesh=mesh,
        out_type=jax.ShapeDtypeStruct((G, FEAT), jnp.float32),
        scratch_types=[
            pltpu.VMEM((PER_W,), jnp.int32),
            pltpu.VMEM((CHUNK, FEAT), jnp.float32),
            pltpu.VMEM((CHUNK, FEAT), jnp.float32),
            pltpu.SemaphoreType.DMA,
            pltpu.SemaphoreType.DMA,
        ],
    )
    def gather_kernel(idx_hbm, feat_hbm, out_hbm, idx_v, buf0, buf1, sem0, sem1):
        wid = lax.axis_index("s") * 2 + lax.axis_index("c")
        base = wid * PER_W
        pltpu.sync_copy(idx_hbm.at[pl.ds(base, PER_W)], idx_v)
        bufs = (buf0, buf1)
        sems = (sem0, sem1)
        pltpu.async_copy(feat_hbm.at[idx_v.at[pl.ds(0, CHUNK)]], bufs[0], sems[0])

        def step(c, carry):
            del carry
            nxt = lax.rem(c + 1, 2)
            cur = lax.rem(c, 2)

            @pl.when(c + 1 < NCH)
            def _():
                @pl.when(nxt == 0)
                def _():
                    pltpu.async_copy(
                        feat_hbm.at[idx_v.at[pl.ds((c + 1) * CHUNK, CHUNK)]],
                        bufs[0], sems[0])

                @pl.when(nxt == 1)
                def _():
                    pltpu.async_copy(
                        feat_hbm.at[idx_v.at[pl.ds((c + 1) * CHUNK, CHUNK)]],
                        bufs[1], sems[1])

            @pl.when(cur == 0)
            def _():
                pltpu.make_async_copy(
                    feat_hbm.at[idx_v.at[pl.ds(c * CHUNK, CHUNK)]],
                    bufs[0], sems[0]).wait()
                pltpu.sync_copy(bufs[0], out_hbm.at[pl.ds(base + c * CHUNK, CHUNK)])

            @pl.when(cur == 1)
            def _():
                pltpu.make_async_copy(
                    feat_hbm.at[idx_v.at[pl.ds(c * CHUNK, CHUNK)]],
                    bufs[1], sems[1]).wait()
                pltpu.sync_copy(bufs[1], out_hbm.at[pl.ds(base + c * CHUNK, CHUNK)])

            return 0

        lax.fori_loop(0, NCH, step, 0)

    return gather_kernel(all_idx, feat)


def _tc_body(nfeat_ref, ffeat_ref, Wn_ref, bn_ref, fW_ref, fb_ref, l2W_ref,
             l2b_ref, Kt_ref, a1t_ref, a1b_ref, a1bias_ref, a2W_ref, a2b_ref,
             a3w_ref, out_ref):
    nfeat = nfeat_ref[...]
    ffeat = ffeat_ref[...]
    nf = jnp.dot(nfeat, Wn_ref[...]) + bn_ref[...]
    ff = jnp.dot(ffeat, fW_ref[...]) + fb_ref[...]

    nfr = jnp.broadcast_to(nf[:, None, :], (BN, F, D)).reshape(BN * F, D)
    cross = nfr * ff

    att = [jnp.sum(cross * Kt_ref[k, :][None, :], axis=1, keepdims=True)
           for k in range(4)]
    m = jnp.maximum(jnp.maximum(att[0], att[1]), jnp.maximum(att[2], att[3]))
    e = [jnp.exp(a - m) for a in att]
    inv = 1.0 / (e[0] + e[1] + e[2] + e[3])

    fil = jnp.zeros((BN * F, D), jnp.float32)
    for k in range(4):
        vk = jnp.maximum(
            jnp.dot(ff, l2W_ref[:, k * D:(k + 1) * D])
            + l2b_ref[:, k * D:(k + 1) * D], 0.0)
        fil = fil + vk * (e[k] * inv)

    nfa = jnp.dot(nf, a1b_ref[...])
    nfa_r = jnp.broadcast_to(nfa[:, None, :], (BN, F, D)).reshape(BN * F, D)
    x = jnp.maximum(jnp.dot(fil, a1t_ref[...]) + nfa_r + a1bias_ref[...], 0.0)
    x = jnp.maximum(jnp.dot(x, a2W_ref[...]) + a2b_ref[...], 0.0)

    x3 = x.reshape(BN, F, D)
    logits = jnp.sum(x3 * a3w_ref[0, :][None, None, :], axis=2)
    mx = jnp.max(logits, axis=1, keepdims=True)
    ew = jnp.exp(logits - mx)
    w = ew / jnp.sum(ew, axis=1, keepdims=True)

    fil3 = fil.reshape(BN, F, D)
    acc = jnp.zeros((BN, D), jnp.float32)
    for f in range(F):
        acc = acc + fil3[:, f, :] * w[:, f][:, None]
    out_ref[...] = acc


def _tc_compute(gathered, Wn, bn, fW, fb, l2_W, l2_b, Kt, a1t, a1b, a1bias,
                a2_W, a2b, a3w):
    rep = lambda shape: pl.BlockSpec(shape, lambda i: tuple(0 for _ in shape))
    return pl.pallas_call(
        _tc_body,
        grid=(B // BN,),
        in_specs=[
            pl.BlockSpec((BN, FEAT), lambda i: (i, 0)),
            pl.BlockSpec((BN * F, FEAT), lambda i: (i + 1, 0)),
            rep((FEAT, D)),
            rep((1, D)),
            rep((FEAT, D)),
            rep((1, D)),
            rep((D, 4 * D)),
            rep((1, 4 * D)),
            rep((4, D)),
            rep((D, D)),
            rep((D, D)),
            rep((1, D)),
            rep((D, D)),
            rep((1, D)),
            rep((1, D)),
        ],
        out_specs=pl.BlockSpec((BN, D), lambda i: (i, 0)),
        out_shape=jax.ShapeDtypeStruct((B, D), jnp.float32),
    )(gathered, gathered, Wn, bn, fW, fb, l2_W, l2_b, Kt, a1t, a1b, a1bias,
      a2_W, a2b, a3w)


def kernel(nodes_u, nodes_i, is_user, friends_table, ufeat, ifeat, u_W, u_b,
           i_W, i_b, uf_W, uf_b, if_W, if_b, l2_W, l2_b, K, a1_W, a1_b, a2_W,
           a2_b, a3_W, a3_b):
    cond = is_user != 0
    nodes = jnp.where(cond, nodes_u, nodes_i)
    feat = lax.cond(cond, lambda: ufeat, lambda: ifeat)
    Wn = jnp.where(cond, u_W, i_W)
    bn = jnp.where(cond, u_b, i_b).reshape(1, D)
    fW = jnp.where(cond, uf_W, if_W)
    fb = jnp.where(cond, uf_b, if_b).reshape(1, D)

    fr_flat = friends_table[nodes].reshape(-1)
    all_idx = jnp.concatenate([nodes, fr_flat])
    gathered = _sc_gather(all_idx, feat)

    del a3_b
    return _tc_compute(
        gathered, Wn, bn, fW, fb, l2_W, l2_b.reshape(1, 4 * D),
        K.T, a1_W[:D, :], a1_W[D:, :], a1_b.reshape(1, D),
        a2_W, a2_b.reshape(1, D), a3_W.reshape(1, D))
